# Initial kernel scaffold; baseline (speedup 1.0000x reference)
import jax
import jax.numpy as jnp
from jax.experimental import pallas as pl


def kernel(x, W1, gamma1, beta1, W2, gamma2, beta2):
    raise NotImplementedError("write your pallas kernel here")


# speedup vs baseline: 6.8199x; 6.8199x over previous
import functools

import jax
import jax.numpy as jnp
from jax import lax
from jax.experimental import pallas as pl

KNN = 20
EPS_BN = 1e-5
NEG = -3.0e38


def _proj_body(xt_ref, w1_ref, u_ref, w_ref, sq_ref, *, C):
    xtb = xt_ref[0]
    w1 = w1_ref[...]
    w1a = w1[:, :C]
    w1d = w1[:, C:] - w1a
    dn = (((1,), (1,)), ((), ()))
    u_ref[0] = lax.dot_general(xtb, w1a, dn, preferred_element_type=jnp.float32)
    w_ref[0] = lax.dot_general(xtb, w1d, dn, preferred_element_type=jnp.float32)
    sq_ref[0, 0, :] = jnp.sum(xtb * xtb, axis=1)


def _topk_body(xt_ref, xtf_ref, sq_ref, idx_ref, *, RB, N, k):
    b = pl.program_id(0)
    xtb = xt_ref[0]
    xtf = xtf_ref[0]
    dn = (((1,), (1,)), ((), ()))
    g = lax.dot_general(xtb, xtf, dn, preferred_element_type=jnp.float32)
    d2 = 2.0 * g - sq_ref[0]
    iota = lax.broadcasted_iota(jnp.int32, (RB, N), 1)
    cols = []
    for _ in range(k):
        m = jnp.max(d2, axis=1, keepdims=True)
        eq = d2 == m
        cand = jnp.where(eq, iota, N)
        amin = jnp.min(cand, axis=1, keepdims=True)
        cols.append(amin)
        d2 = jnp.where(cand == amin, NEG, d2)
    idx = jnp.concatenate(cols, axis=1)
    idx_ref[0] = idx + b * N


def _stats1_body(g_ref, w_ref, st_ref):
    first = (pl.program_id(0) == 0) & (pl.program_id(1) == 0)
    y1 = g_ref[0] + w_ref[0][:, None, :]
    s = jnp.sum(y1, axis=(0, 1))
    ss = jnp.sum(y1 * y1, axis=(0, 1))
    st = jnp.concatenate([s[None, :], ss[None, :]], axis=0)

    @pl.when(first)
    def _():
        st_ref[...] = st

    @pl.when(jnp.logical_not(first))
    def _():
        st_ref[...] = st_ref[...] + st


def _main_body(g_ref, w_ref, st1_ref, g1_ref, b1_ref, w2_ref,
               ymax_ref, ymin_ref, st2_ref, *, RB, k, O, M):
    first = (pl.program_id(0) == 0) & (pl.program_id(1) == 0)
    mean = st1_ref[0, :] * (1.0 / M)
    var = st1_ref[1, :] * (1.0 / M) - mean * mean
    inv = lax.rsqrt(var + EPS_BN)
    a1 = g1_ref[0] * inv
    c1 = b1_ref[0] - mean * a1
    y1 = g_ref[0] + w_ref[0][:, None, :]
    z = y1 * a1[None, None, :] + c1[None, None, :]
    z = jnp.where(z >= 0, z, 0.2 * z)
    zf = z.reshape(RB * k, O)
    dn = (((1,), (1,)), ((), ()))
    y2 = lax.dot_general(zf, w2_ref[...], dn, preferred_element_type=jnp.float32)
    s = jnp.sum(y2, axis=0)
    ss = jnp.sum(y2 * y2, axis=0)
    st = jnp.concatenate([s[None, :], ss[None, :]], axis=0)
    y2r = y2.reshape(RB, k, O)
    ymax_ref[0] = jnp.max(y2r, axis=1)
    ymin_ref[0] = jnp.min(y2r, axis=1)

    @pl.when(first)
    def _():
        st2_ref[...] = st

    @pl.when(jnp.logical_not(first))
    def _():
        st2_ref[...] = st2_ref[...] + st


def _final_body(ymax_ref, ymin_ref, st2_ref, g2_ref, b2_ref, o_ref, *, M):
    mean = st2_ref[0, :] * (1.0 / M)
    var = st2_ref[1, :] * (1.0 / M) - mean * mean
    inv = lax.rsqrt(var + EPS_BN)
    a2 = g2_ref[0] * inv
    c2 = b2_ref[0] - mean * a2
    pick = jnp.where(a2[None, :] >= 0, ymax_ref[0], ymin_ref[0])
    v = pick * a2[None, :] + c2[None, :]
    o_ref[0] = jnp.where(v >= 0, v, 0.2 * v)


def kernel(x, W1, gamma1, beta1, W2, gamma2, beta2):
    B, C, N = x.shape
    O = W1.shape[0]
    k = KNN
    RB = 256 if N % 256 == 0 else N
    NB = N // RB
    M = float(B * N * k)
    f32 = jnp.float32

    xt = jnp.transpose(x, (0, 2, 1))

    u, w, sq = pl.pallas_call(
        functools.partial(_proj_body, C=C),
        grid=(B, NB),
        in_specs=[
            pl.BlockSpec((1, RB, C), lambda b, r: (b, r, 0)),
            pl.BlockSpec((O, 2 * C), lambda b, r: (0, 0)),
        ],
        out_specs=[
            pl.BlockSpec((1, RB, O), lambda b, r: (b, r, 0)),
            pl.BlockSpec((1, RB, O), lambda b, r: (b, r, 0)),
            pl.BlockSpec((1, 1, RB), lambda b, r: (b, 0, r)),
        ],
        out_shape=[
            jax.ShapeDtypeStruct((B, N, O), f32),
            jax.ShapeDtypeStruct((B, N, O), f32),
            jax.ShapeDtypeStruct((B, 1, N), f32),
        ],
    )(xt, W1)

    idx = pl.pallas_call(
        functools.partial(_topk_body, RB=RB, N=N, k=k),
        grid=(B, NB),
        in_specs=[
            pl.BlockSpec((1, RB, C), lambda b, r: (b, r, 0)),
            pl.BlockSpec((1, N, C), lambda b, r: (b, 0, 0)),
            pl.BlockSpec((1, 1, N), lambda b, r: (b, 0, 0)),
        ],
        out_specs=pl.BlockSpec((1, RB, k), lambda b, r: (b, r, 0)),
        out_shape=jax.ShapeDtypeStruct((B, N, k), jnp.int32),
    )(xt, xt, sq)

    g = u.reshape(B * N, O)[idx.reshape(-1)].reshape(B, N, k, O)

    st1 = pl.pallas_call(
        _stats1_body,
        grid=(B, NB),
        in_specs=[
            pl.BlockSpec((1, RB, k, O), lambda b, r: (b, r, 0, 0)),
            pl.BlockSpec((1, RB, O), lambda b, r: (b, r, 0)),
        ],
        out_specs=pl.BlockSpec((2, O), lambda b, r: (0, 0)),
        out_shape=jax.ShapeDtypeStruct((2, O), f32),
    )(g, w)

    g1 = gamma1.reshape(1, O)
    b1 = beta1.reshape(1, O)

    ymax, ymin, st2 = pl.pallas_call(
        functools.partial(_main_body, RB=RB, k=k, O=O, M=M),
        grid=(B, NB),
        in_specs=[
            pl.BlockSpec((1, RB, k, O), lambda b, r: (b, r, 0, 0)),
            pl.BlockSpec((1, RB, O), lambda b, r: (b, r, 0)),
            pl.BlockSpec((2, O), lambda b, r: (0, 0)),
            pl.BlockSpec((1, O), lambda b, r: (0, 0)),
            pl.BlockSpec((1, O), lambda b, r: (0, 0)),
            pl.BlockSpec((O, O), lambda b, r: (0, 0)),
        ],
        out_specs=[
            pl.BlockSpec((1, RB, O), lambda b, r: (b, r, 0)),
            pl.BlockSpec((1, RB, O), lambda b, r: (b, r, 0)),
            pl.BlockSpec((2, O), lambda b, r: (0, 0)),
        ],
        out_shape=[
            jax.ShapeDtypeStruct((B, N, O), f32),
            jax.ShapeDtypeStruct((B, N, O), f32),
            jax.ShapeDtypeStruct((2, O), f32),
        ],
    )(g, w, st1, g1, b1, W2)

    g2 = gamma2.reshape(1, O)
    b2 = beta2.reshape(1, O)

    o = pl.pallas_call(
        functools.partial(_final_body, M=M),
        grid=(B, NB),
        in_specs=[
            pl.BlockSpec((1, RB, O), lambda b, r: (b, r, 0)),
            pl.BlockSpec((1, RB, O), lambda b, r: (b, r, 0)),
            pl.BlockSpec((2, O), lambda b, r: (0, 0)),
            pl.BlockSpec((1, O), lambda b, r: (0, 0)),
            pl.BlockSpec((1, O), lambda b, r: (0, 0)),
        ],
        out_specs=pl.BlockSpec((1, RB, O), lambda b, r: (b, r, 0)),
        out_shape=jax.ShapeDtypeStruct((B, N, O), f32),
    )(ymax, ymin, st2, g2, b2)

    return jnp.transpose(o, (0, 2, 1))


# speedup vs baseline: 9.7019x; 1.4226x over previous
import functools

import jax
import jax.numpy as jnp
from jax import lax
from jax.experimental import pallas as pl
from jax.experimental.pallas import tpu as pltpu
from jax.experimental.pallas import tpu_sc as plsc

KNN = 20
EPS_BN = 1e-5
NEG = -3.0e38


def _proj_body(xt_ref, w1_ref, u_ref, w_ref, sq_ref, *, C, O, OP):
    xtb = xt_ref[0]
    w1 = w1_ref[...]
    w1a = w1[:, :C]
    w1d = w1[:, C:] - w1a
    dn = (((1,), (1,)), ((), ()))
    ub = lax.dot_general(xtb, w1a, dn, preferred_element_type=jnp.float32)
    if OP > O:
        ub = jnp.concatenate(
            [ub, jnp.zeros((ub.shape[0], OP - O), jnp.float32)], axis=1)
    u_ref[0] = ub
    w_ref[0] = lax.dot_general(xtb, w1d, dn, preferred_element_type=jnp.float32)
    sq_ref[0, 0, :] = jnp.sum(xtb * xtb, axis=1)


def _topk_body(xt_ref, xtf_ref, sq_ref, idx_ref, *, RB, N, k):
    b = pl.program_id(0)
    xtb = xt_ref[0]
    xtf = xtf_ref[0]
    dn = (((1,), (1,)), ((), ()))
    g = lax.dot_general(xtb, xtf, dn, preferred_element_type=jnp.float32)
    d2 = 2.0 * g - sq_ref[0]
    iota = lax.broadcasted_iota(jnp.int32, (RB, N), 1)
    cols = []
    for _ in range(k):
        m = jnp.max(d2, axis=1, keepdims=True)
        eq = d2 == m
        cand = jnp.where(eq, iota, N)
        amin = jnp.min(cand, axis=1, keepdims=True)
        cols.append(amin)
        d2 = jnp.where(cand == amin, NEG, d2)
    idx = jnp.concatenate(cols, axis=1)
    idx_ref[0] = idx + b * N


def _stats1_body(g_ref, w_ref, st_ref, *, O):
    first = (pl.program_id(0) == 0) & (pl.program_id(1) == 0)
    y1 = g_ref[0][:, :, :O] + w_ref[0][:, None, :]
    s = jnp.sum(y1, axis=(0, 1))
    ss = jnp.sum(y1 * y1, axis=(0, 1))
    st = jnp.concatenate([s[None, :], ss[None, :]], axis=0)

    @pl.when(first)
    def _():
        st_ref[...] = st

    @pl.when(jnp.logical_not(first))
    def _():
        st_ref[...] = st_ref[...] + st


def _main_body(g_ref, w_ref, st1_ref, g1_ref, b1_ref, w2_ref,
               ymax_ref, ymin_ref, st2_ref, *, RB, k, O, M):
    first = (pl.program_id(0) == 0) & (pl.program_id(1) == 0)
    mean = st1_ref[0, :] * (1.0 / M)
    var = st1_ref[1, :] * (1.0 / M) - mean * mean
    inv = lax.rsqrt(var + EPS_BN)
    a1 = g1_ref[0] * inv
    c1 = b1_ref[0] - mean * a1
    y1 = g_ref[0][:, :, :O] + w_ref[0][:, None, :]
    z = y1 * a1[None, None, :] + c1[None, None, :]
    z = jnp.where(z >= 0, z, 0.2 * z)
    zf = z.reshape(RB * k, O)
    dn = (((1,), (1,)), ((), ()))
    y2 = lax.dot_general(zf, w2_ref[...], dn, preferred_element_type=jnp.float32)
    s = jnp.sum(y2, axis=0)
    ss = jnp.sum(y2 * y2, axis=0)
    st = jnp.concatenate([s[None, :], ss[None, :]], axis=0)
    y2r = y2.reshape(RB, k, O)
    ymax_ref[0] = jnp.max(y2r, axis=1)
    ymin_ref[0] = jnp.min(y2r, axis=1)

    @pl.when(first)
    def _():
        st2_ref[...] = st

    @pl.when(jnp.logical_not(first))
    def _():
        st2_ref[...] = st2_ref[...] + st


def _final_body(ymax_ref, ymin_ref, st2_ref, g2_ref, b2_ref, o_ref, *, M):
    mean = st2_ref[0, :] * (1.0 / M)
    var = st2_ref[1, :] * (1.0 / M) - mean * mean
    inv = lax.rsqrt(var + EPS_BN)
    a2 = g2_ref[0] * inv
    c2 = b2_ref[0] - mean * a2
    pick = jnp.where(a2[None, :] >= 0, ymax_ref[0], ymin_ref[0])
    v = pick * a2[None, :] + c2[None, :]
    o_ref[0] = jnp.where(v >= 0, v, 0.2 * v)


def _sc_gather(table, idx_flat, O):
    tot = idx_flat.shape[0]
    info = plsc.get_sparse_core_info()
    nc, ns = info.num_cores, info.num_subcores
    nw = nc * ns
    per_w = tot // nw
    ch = min(512, per_w)
    nch = per_w // ch
    mesh = plsc.VectorSubcoreMesh(core_axis_name="c", subcore_axis_name="s")

    @functools.partial(
        pl.kernel,
        mesh=mesh,
        out_type=jax.ShapeDtypeStruct((tot, O), jnp.float32),
        scratch_types=[
            pltpu.VMEM((ch,), jnp.int32),
            pltpu.VMEM((ch, O), jnp.float32),
            pltpu.SemaphoreType.DMA,
        ],
    )
    def gk(table_hbm, idx_hbm, out_hbm, idx_v, rows_v, sem):
        wid = lax.axis_index("s") * nc + lax.axis_index("c")

        def body(i, carry):
            base = wid * per_w + i * ch
            pltpu.sync_copy(idx_hbm.at[pl.ds(base, ch)], idx_v)
            pltpu.async_copy(table_hbm.at[idx_v], rows_v, sem).wait()
            pltpu.sync_copy(rows_v, out_hbm.at[pl.ds(base, ch)])
            return carry

        lax.fori_loop(0, nch, body, 0)

    return gk(table, idx_flat)


def kernel(x, W1, gamma1, beta1, W2, gamma2, beta2):
    B, C, N = x.shape
    O = W1.shape[0]
    k = KNN
    RB = 256 if N % 256 == 0 else N
    NB = N // RB
    M = float(B * N * k)
    f32 = jnp.float32

    xt = jnp.transpose(x, (0, 2, 1))

    OP = O if O % 128 == 0 else ((O // 128) + 1) * 128

    u, w, sq = pl.pallas_call(
        functools.partial(_proj_body, C=C, O=O, OP=OP),
        grid=(B, NB),
        in_specs=[
            pl.BlockSpec((1, RB, C), lambda b, r: (b, r, 0)),
            pl.BlockSpec((O, 2 * C), lambda b, r: (0, 0)),
        ],
        out_specs=[
            pl.BlockSpec((1, RB, OP), lambda b, r: (b, r, 0)),
            pl.BlockSpec((1, RB, O), lambda b, r: (b, r, 0)),
            pl.BlockSpec((1, 1, RB), lambda b, r: (b, 0, r)),
        ],
        out_shape=[
            jax.ShapeDtypeStruct((B, N, OP), f32),
            jax.ShapeDtypeStruct((B, N, O), f32),
            jax.ShapeDtypeStruct((B, 1, N), f32),
        ],
    )(xt, W1)

    idx = pl.pallas_call(
        functools.partial(_topk_body, RB=RB, N=N, k=k),
        grid=(B, NB),
        in_specs=[
            pl.BlockSpec((1, RB, C), lambda b, r: (b, r, 0)),
            pl.BlockSpec((1, N, C), lambda b, r: (b, 0, 0)),
            pl.BlockSpec((1, 1, N), lambda b, r: (b, 0, 0)),
        ],
        out_specs=pl.BlockSpec((1, RB, k), lambda b, r: (b, r, 0)),
        out_shape=jax.ShapeDtypeStruct((B, N, k), jnp.int32),
    )(xt, xt, sq)

    g = _sc_gather(u.reshape(B * N, OP), idx.reshape(-1), OP).reshape(B, N, k, OP)

    st1 = pl.pallas_call(
        functools.partial(_stats1_body, O=O),
        grid=(B, NB),
        in_specs=[
            pl.BlockSpec((1, RB, k, OP), lambda b, r: (b, r, 0, 0)),
            pl.BlockSpec((1, RB, O), lambda b, r: (b, r, 0)),
        ],
        out_specs=pl.BlockSpec((2, O), lambda b, r: (0, 0)),
        out_shape=jax.ShapeDtypeStruct((2, O), f32),
    )(g, w)

    g1 = gamma1.reshape(1, O)
    b1 = beta1.reshape(1, O)

    ymax, ymin, st2 = pl.pallas_call(
        functools.partial(_main_body, RB=RB, k=k, O=O, M=M),
        grid=(B, NB),
        in_specs=[
            pl.BlockSpec((1, RB, k, OP), lambda b, r: (b, r, 0, 0)),
            pl.BlockSpec((1, RB, O), lambda b, r: (b, r, 0)),
            pl.BlockSpec((2, O), lambda b, r: (0, 0)),
            pl.BlockSpec((1, O), lambda b, r: (0, 0)),
            pl.BlockSpec((1, O), lambda b, r: (0, 0)),
            pl.BlockSpec((O, O), lambda b, r: (0, 0)),
        ],
        out_specs=[
            pl.BlockSpec((1, RB, O), lambda b, r: (b, r, 0)),
            pl.BlockSpec((1, RB, O), lambda b, r: (b, r, 0)),
            pl.BlockSpec((2, O), lambda b, r: (0, 0)),
        ],
        out_shape=[
            jax.ShapeDtypeStruct((B, N, O), f32),
            jax.ShapeDtypeStruct((B, N, O), f32),
            jax.ShapeDtypeStruct((2, O), f32),
        ],
    )(g, w, st1, g1, b1, W2)

    g2 = gamma2.reshape(1, O)
    b2 = beta2.reshape(1, O)

    o = pl.pallas_call(
        functools.partial(_final_body, M=M),
        grid=(B, NB),
        in_specs=[
            pl.BlockSpec((1, RB, O), lambda b, r: (b, r, 0)),
            pl.BlockSpec((1, RB, O), lambda b, r: (b, r, 0)),
            pl.BlockSpec((2, O), lambda b, r: (0, 0)),
            pl.BlockSpec((1, O), lambda b, r: (0, 0)),
            pl.BlockSpec((1, O), lambda b, r: (0, 0)),
        ],
        out_specs=pl.BlockSpec((1, RB, O), lambda b, r: (b, r, 0)),
        out_shape=jax.ShapeDtypeStruct((B, N, O), f32),
    )(ymax, ymin, st2, g2, b2)

    return jnp.transpose(o, (0, 2, 1))


# speedup vs baseline: 11.3869x; 1.1737x over previous
import functools

import jax
import jax.numpy as jnp
from jax import lax
from jax.experimental import pallas as pl
from jax.experimental.pallas import tpu as pltpu
from jax.experimental.pallas import tpu_sc as plsc

KNN = 20
EPS_BN = 1e-5
NEG = -3.0e38


def _proj_body(xt_ref, w1_ref, u_ref, w_ref, sq_ref, *, C, O, OP):
    xtb = xt_ref[0]
    w1 = w1_ref[...]
    w1a = w1[:, :C]
    w1d = w1[:, C:] - w1a
    dn = (((1,), (1,)), ((), ()))
    ub = lax.dot_general(xtb, w1a, dn, preferred_element_type=jnp.float32)
    if OP > O:
        ub = jnp.concatenate(
            [ub, jnp.zeros((ub.shape[0], OP - O), jnp.float32)], axis=1)
    u_ref[0] = ub
    w_ref[0] = lax.dot_general(xtb, w1d, dn, preferred_element_type=jnp.float32)
    sq_ref[0, 0, :] = jnp.sum(xtb * xtb, axis=1)


def _topk_body(xt_ref, xtf_ref, sq_ref, idx_ref, *, RB, N, k):
    b = pl.program_id(0)
    xtb = xt_ref[0]
    xtf = xtf_ref[0]
    dn = (((1,), (1,)), ((), ()))
    g = lax.dot_general(xtb, xtf, dn, preferred_element_type=jnp.float32)
    d2 = 2.0 * g - sq_ref[0]
    iota = lax.broadcasted_iota(jnp.int32, (RB, N), 1)
    r = pl.program_id(1)
    selfc = r * RB + lax.broadcasted_iota(jnp.int32, (RB, 1), 0)
    cols = [selfc]
    d2 = jnp.where(iota == selfc, NEG, d2)
    for _ in range(k - 1):
        aj = jnp.argmax(d2, axis=1).astype(jnp.int32)[:, None]
        cols.append(aj)
        d2 = jnp.where(iota == aj, NEG, d2)
    idx = jnp.concatenate(cols, axis=1)
    idx_ref[0] = idx + b * N


def _stats1_body(g_ref, w_ref, st_ref, *, O):
    first = (pl.program_id(0) == 0) & (pl.program_id(1) == 0)
    y1 = g_ref[0][:, :, :O] + w_ref[0][:, None, :]
    s = jnp.sum(y1, axis=(0, 1))
    ss = jnp.sum(y1 * y1, axis=(0, 1))
    st = jnp.concatenate([s[None, :], ss[None, :]], axis=0)

    @pl.when(first)
    def _():
        st_ref[...] = st

    @pl.when(jnp.logical_not(first))
    def _():
        st_ref[...] = st_ref[...] + st


def _main_body(g_ref, w_ref, st1_ref, g1_ref, b1_ref, w2_ref,
               ymax_ref, ymin_ref, st2_ref, *, RB, k, O, M):
    first = (pl.program_id(0) == 0) & (pl.program_id(1) == 0)
    mean = st1_ref[0, :] * (1.0 / M)
    var = st1_ref[1, :] * (1.0 / M) - mean * mean
    inv = lax.rsqrt(var + EPS_BN)
    a1 = g1_ref[0] * inv
    c1 = b1_ref[0] - mean * a1
    y1 = g_ref[0][:, :, :O] + w_ref[0][:, None, :]
    z = y1 * a1[None, None, :] + c1[None, None, :]
    z = jnp.where(z >= 0, z, 0.2 * z)
    zf = z.reshape(RB * k, O)
    dn = (((1,), (1,)), ((), ()))
    y2 = lax.dot_general(zf, w2_ref[...], dn, preferred_element_type=jnp.float32)
    s = jnp.sum(y2, axis=0)
    ss = jnp.sum(y2 * y2, axis=0)
    st = jnp.concatenate([s[None, :], ss[None, :]], axis=0)
    y2r = y2.reshape(RB, k, O)
    ymax_ref[0] = jnp.max(y2r, axis=1)
    ymin_ref[0] = jnp.min(y2r, axis=1)

    @pl.when(first)
    def _():
        st2_ref[...] = st

    @pl.when(jnp.logical_not(first))
    def _():
        st2_ref[...] = st2_ref[...] + st


def _final_body(ymax_ref, ymin_ref, st2_ref, g2_ref, b2_ref, o_ref, *, M):
    mean = st2_ref[0, :] * (1.0 / M)
    var = st2_ref[1, :] * (1.0 / M) - mean * mean
    inv = lax.rsqrt(var + EPS_BN)
    a2 = g2_ref[0] * inv
    c2 = b2_ref[0] - mean * a2
    pick = jnp.where(a2[None, :] >= 0, ymax_ref[0], ymin_ref[0])
    v = pick * a2[None, :] + c2[None, :]
    o_ref[0] = jnp.where(v >= 0, v, 0.2 * v)


def _sc_gather(table, idx_flat, O):
    tot = idx_flat.shape[0]
    info = plsc.get_sparse_core_info()
    nc, ns = info.num_cores, info.num_subcores
    nw = nc * ns
    per_w = tot // nw
    ch = min(512, per_w)
    nch = per_w // ch
    mesh = plsc.VectorSubcoreMesh(core_axis_name="c", subcore_axis_name="s")

    @functools.partial(
        pl.kernel,
        mesh=mesh,
        out_type=jax.ShapeDtypeStruct((tot, O), jnp.float32),
        scratch_types=[
            pltpu.VMEM((ch,), jnp.int32),
            pltpu.VMEM((ch, O), jnp.float32),
            pltpu.SemaphoreType.DMA,
        ],
    )
    def gk(table_hbm, idx_hbm, out_hbm, idx_v, rows_v, sem):
        wid = lax.axis_index("s") * nc + lax.axis_index("c")

        def body(i, carry):
            base = wid * per_w + i * ch
            pltpu.sync_copy(idx_hbm.at[pl.ds(base, ch)], idx_v)
            pltpu.async_copy(table_hbm.at[idx_v], rows_v, sem).wait()
            pltpu.sync_copy(rows_v, out_hbm.at[pl.ds(base, ch)])
            return carry

        lax.fori_loop(0, nch, body, 0)

    return gk(table, idx_flat)


def kernel(x, W1, gamma1, beta1, W2, gamma2, beta2):
    B, C, N = x.shape
    O = W1.shape[0]
    k = KNN
    RB = 256 if N % 256 == 0 else N
    NB = N // RB
    M = float(B * N * k)
    f32 = jnp.float32

    xt = jnp.transpose(x, (0, 2, 1))

    OP = O if O % 128 == 0 else ((O // 128) + 1) * 128

    u, w, sq = pl.pallas_call(
        functools.partial(_proj_body, C=C, O=O, OP=OP),
        grid=(B, NB),
        in_specs=[
            pl.BlockSpec((1, RB, C), lambda b, r: (b, r, 0)),
            pl.BlockSpec((O, 2 * C), lambda b, r: (0, 0)),
        ],
        out_specs=[
            pl.BlockSpec((1, RB, OP), lambda b, r: (b, r, 0)),
            pl.BlockSpec((1, RB, O), lambda b, r: (b, r, 0)),
            pl.BlockSpec((1, 1, RB), lambda b, r: (b, 0, r)),
        ],
        out_shape=[
            jax.ShapeDtypeStruct((B, N, OP), f32),
            jax.ShapeDtypeStruct((B, N, O), f32),
            jax.ShapeDtypeStruct((B, 1, N), f32),
        ],
    )(xt, W1)

    idx = pl.pallas_call(
        functools.partial(_topk_body, RB=RB, N=N, k=k),
        grid=(B, NB),
        in_specs=[
            pl.BlockSpec((1, RB, C), lambda b, r: (b, r, 0)),
            pl.BlockSpec((1, N, C), lambda b, r: (b, 0, 0)),
            pl.BlockSpec((1, 1, N), lambda b, r: (b, 0, 0)),
        ],
        out_specs=pl.BlockSpec((1, RB, k), lambda b, r: (b, r, 0)),
        out_shape=jax.ShapeDtypeStruct((B, N, k), jnp.int32),
    )(xt, xt, sq)

    g = _sc_gather(u.reshape(B * N, OP), idx.reshape(-1), OP).reshape(B, N, k, OP)

    st1 = pl.pallas_call(
        functools.partial(_stats1_body, O=O),
        grid=(B, NB),
        in_specs=[
            pl.BlockSpec((1, RB, k, OP), lambda b, r: (b, r, 0, 0)),
            pl.BlockSpec((1, RB, O), lambda b, r: (b, r, 0)),
        ],
        out_specs=pl.BlockSpec((2, O), lambda b, r: (0, 0)),
        out_shape=jax.ShapeDtypeStruct((2, O), f32),
    )(g, w)

    g1 = gamma1.reshape(1, O)
    b1 = beta1.reshape(1, O)

    ymax, ymin, st2 = pl.pallas_call(
        functools.partial(_main_body, RB=RB, k=k, O=O, M=M),
        grid=(B, NB),
        in_specs=[
            pl.BlockSpec((1, RB, k, OP), lambda b, r: (b, r, 0, 0)),
            pl.BlockSpec((1, RB, O), lambda b, r: (b, r, 0)),
            pl.BlockSpec((2, O), lambda b, r: (0, 0)),
            pl.BlockSpec((1, O), lambda b, r: (0, 0)),
            pl.BlockSpec((1, O), lambda b, r: (0, 0)),
            pl.BlockSpec((O, O), lambda b, r: (0, 0)),
        ],
        out_specs=[
            pl.BlockSpec((1, RB, O), lambda b, r: (b, r, 0)),
            pl.BlockSpec((1, RB, O), lambda b, r: (b, r, 0)),
            pl.BlockSpec((2, O), lambda b, r: (0, 0)),
        ],
        out_shape=[
            jax.ShapeDtypeStruct((B, N, O), f32),
            jax.ShapeDtypeStruct((B, N, O), f32),
            jax.ShapeDtypeStruct((2, O), f32),
        ],
    )(g, w, st1, g1, b1, W2)

    g2 = gamma2.reshape(1, O)
    b2 = beta2.reshape(1, O)

    o = pl.pallas_call(
        functools.partial(_final_body, M=M),
        grid=(B, NB),
        in_specs=[
            pl.BlockSpec((1, RB, O), lambda b, r: (b, r, 0)),
            pl.BlockSpec((1, RB, O), lambda b, r: (b, r, 0)),
            pl.BlockSpec((2, O), lambda b, r: (0, 0)),
            pl.BlockSpec((1, O), lambda b, r: (0, 0)),
            pl.BlockSpec((1, O), lambda b, r: (0, 0)),
        ],
        out_specs=pl.BlockSpec((1, RB, O), lambda b, r: (b, r, 0)),
        out_shape=jax.ShapeDtypeStruct((B, N, O), f32),
    )(ymax, ymin, st2, g2, b2)

    return jnp.transpose(o, (0, 2, 1))


# speedup vs baseline: 11.3975x; 1.0009x over previous
import functools

import jax
import jax.numpy as jnp
from jax import lax
from jax.experimental import pallas as pl
from jax.experimental.pallas import tpu as pltpu
from jax.experimental.pallas import tpu_sc as plsc

KNN = 20
EPS_BN = 1e-5
NEG = -3.0e38


def _proj_body(xt_ref, w1_ref, u_ref, w_ref, sq_ref, *, C, O, OP):
    xtb = xt_ref[0]
    w1 = w1_ref[...]
    w1a = w1[:, :C]
    w1d = w1[:, C:] - w1a
    dn = (((1,), (1,)), ((), ()))
    ub = lax.dot_general(xtb, w1a, dn, preferred_element_type=jnp.float32)
    if OP > O:
        ub = jnp.concatenate(
            [ub, jnp.zeros((ub.shape[0], OP - O), jnp.float32)], axis=1)
    u_ref[0] = ub
    w_ref[0] = lax.dot_general(xtb, w1d, dn, preferred_element_type=jnp.float32)
    sq_ref[0, 0, :] = jnp.sum(xtb * xtb, axis=1)


def _topk_body(xt_ref, xtf_ref, sq_ref, idx_ref, *, RB, N, k):
    b = pl.program_id(0)
    xtb = xt_ref[0]
    xtf = xtf_ref[0]
    dn = (((1,), (1,)), ((), ()))
    g = lax.dot_general(xtb, xtf, dn, preferred_element_type=jnp.float32)
    d2 = 2.0 * g - sq_ref[0]
    iota = lax.broadcasted_iota(jnp.int32, (RB, N), 1)
    r = pl.program_id(1)
    selfc = r * RB + lax.broadcasted_iota(jnp.int32, (RB, 1), 0)
    cols = [selfc]
    d2 = jnp.where(iota == selfc, NEG, d2)
    for _ in range(k - 1):
        aj = jnp.argmax(d2, axis=1).astype(jnp.int32)[:, None]
        cols.append(aj)
        d2 = jnp.where(iota == aj, NEG, d2)
    idx = jnp.concatenate(cols, axis=1)
    idx_ref[0] = idx + b * N


def _stats1_body(g_ref, w_ref, st_ref, *, O):
    first = (pl.program_id(0) == 0) & (pl.program_id(1) == 0)
    y1 = g_ref[0][:, :, :O] + w_ref[0][:, None, :]
    s = jnp.sum(y1, axis=(0, 1))
    ss = jnp.sum(y1 * y1, axis=(0, 1))
    st = jnp.concatenate([s[None, :], ss[None, :]], axis=0)

    @pl.when(first)
    def _():
        st_ref[...] = st

    @pl.when(jnp.logical_not(first))
    def _():
        st_ref[...] = st_ref[...] + st


def _main_body(g_ref, w_ref, st1_ref, g1_ref, b1_ref, w2_ref,
               ymax_ref, ymin_ref, st2_ref, *, RB, k, O, M):
    first = (pl.program_id(0) == 0) & (pl.program_id(1) == 0)
    mean = st1_ref[0, :] * (1.0 / M)
    var = st1_ref[1, :] * (1.0 / M) - mean * mean
    inv = lax.rsqrt(var + EPS_BN)
    a1 = g1_ref[0] * inv
    c1 = b1_ref[0] - mean * a1
    y1 = g_ref[0][:, :, :O] + w_ref[0][:, None, :]
    z = y1 * a1[None, None, :] + c1[None, None, :]
    z = jnp.where(z >= 0, z, 0.2 * z)
    zf = z.reshape(RB * k, O)
    dn = (((1,), (1,)), ((), ()))
    y2 = lax.dot_general(zf, w2_ref[...], dn, preferred_element_type=jnp.float32)
    s = jnp.sum(y2, axis=0)
    ss = jnp.sum(y2 * y2, axis=0)
    st = jnp.concatenate([s[None, :], ss[None, :]], axis=0)
    y2r = y2.reshape(RB, k, O)
    ymax_ref[0] = jnp.max(y2r, axis=1)
    ymin_ref[0] = jnp.min(y2r, axis=1)

    @pl.when(first)
    def _():
        st2_ref[...] = st

    @pl.when(jnp.logical_not(first))
    def _():
        st2_ref[...] = st2_ref[...] + st


def _final_body(ymax_ref, ymin_ref, st2_ref, g2_ref, b2_ref, o_ref, *, M):
    mean = st2_ref[0, :] * (1.0 / M)
    var = st2_ref[1, :] * (1.0 / M) - mean * mean
    inv = lax.rsqrt(var + EPS_BN)
    a2 = g2_ref[0] * inv
    c2 = b2_ref[0] - mean * a2
    pick = jnp.where(a2[None, :] >= 0, ymax_ref[0], ymin_ref[0])
    v = pick * a2[None, :] + c2[None, :]
    o_ref[0] = jnp.where(v >= 0, v, 0.2 * v)


def _sc_gather(table, idx_flat, O):
    tot = idx_flat.shape[0]
    info = plsc.get_sparse_core_info()
    nc, ns = info.num_cores, info.num_subcores
    nw = nc * ns
    per_w = tot // nw
    ch = min(320, per_w)
    nch = per_w // ch
    mesh = plsc.VectorSubcoreMesh(core_axis_name="c", subcore_axis_name="s")

    @functools.partial(
        pl.kernel,
        mesh=mesh,
        out_type=jax.ShapeDtypeStruct((tot, O), jnp.float32),
        scratch_types=[
            pltpu.VMEM((per_w,), jnp.int32),
            pltpu.VMEM((ch, O), jnp.float32),
            pltpu.VMEM((ch, O), jnp.float32),
            pltpu.SemaphoreType.DMA,
            pltpu.SemaphoreType.DMA,
        ],
    )
    def gk(table_hbm, idx_hbm, out_hbm, idx_v, rows0, rows1, sem0, sem1):
        wid = lax.axis_index("s") * nc + lax.axis_index("c")
        base_w = wid * per_w
        pltpu.sync_copy(idx_hbm.at[pl.ds(base_w, per_w)], idx_v)

        def start(i, rows, sem):
            off = pl.multiple_of(i * ch, 8)
            pltpu.async_copy(table_hbm.at[idx_v.at[pl.ds(off, ch)]], rows, sem)

        def drain(rows, sem):
            pltpu.make_async_copy(table_hbm.at[pl.ds(0, ch)], rows, sem).wait()

        def wb(i, rows):
            off = pl.multiple_of(base_w + i * ch, 8)
            pltpu.sync_copy(rows, out_hbm.at[pl.ds(off, ch)])

        start(0, rows0, sem0)

        def body(t, carry):
            i0 = 2 * t
            start(i0 + 1, rows1, sem1)
            drain(rows0, sem0)
            wb(i0, rows0)

            @pl.when(t < (nch // 2) - 1)
            def _():
                start(i0 + 2, rows0, sem0)

            drain(rows1, sem1)
            wb(i0 + 1, rows1)
            return carry

        lax.fori_loop(0, nch // 2, body, 0)

    return gk(table, idx_flat)


def kernel(x, W1, gamma1, beta1, W2, gamma2, beta2):
    B, C, N = x.shape
    O = W1.shape[0]
    k = KNN
    RB = 256 if N % 256 == 0 else N
    NB = N // RB
    M = float(B * N * k)
    f32 = jnp.float32

    xt = jnp.transpose(x, (0, 2, 1))

    OP = O if O % 128 == 0 else ((O // 128) + 1) * 128

    u, w, sq = pl.pallas_call(
        functools.partial(_proj_body, C=C, O=O, OP=OP),
        grid=(B, NB),
        in_specs=[
            pl.BlockSpec((1, RB, C), lambda b, r: (b, r, 0)),
            pl.BlockSpec((O, 2 * C), lambda b, r: (0, 0)),
        ],
        out_specs=[
            pl.BlockSpec((1, RB, OP), lambda b, r: (b, r, 0)),
            pl.BlockSpec((1, RB, O), lambda b, r: (b, r, 0)),
            pl.BlockSpec((1, 1, RB), lambda b, r: (b, 0, r)),
        ],
        out_shape=[
            jax.ShapeDtypeStruct((B, N, OP), f32),
            jax.ShapeDtypeStruct((B, N, O), f32),
            jax.ShapeDtypeStruct((B, 1, N), f32),
        ],
    )(xt, W1)

    idx = pl.pallas_call(
        functools.partial(_topk_body, RB=RB, N=N, k=k),
        grid=(B, NB),
        in_specs=[
            pl.BlockSpec((1, RB, C), lambda b, r: (b, r, 0)),
            pl.BlockSpec((1, N, C), lambda b, r: (b, 0, 0)),
            pl.BlockSpec((1, 1, N), lambda b, r: (b, 0, 0)),
        ],
        out_specs=pl.BlockSpec((1, RB, k), lambda b, r: (b, r, 0)),
        out_shape=jax.ShapeDtypeStruct((B, N, k), jnp.int32),
    )(xt, xt, sq)

    g = _sc_gather(u.reshape(B * N, OP), idx.reshape(-1), OP).reshape(B, N, k, OP)

    st1 = pl.pallas_call(
        functools.partial(_stats1_body, O=O),
        grid=(B, NB),
        in_specs=[
            pl.BlockSpec((1, RB, k, OP), lambda b, r: (b, r, 0, 0)),
            pl.BlockSpec((1, RB, O), lambda b, r: (b, r, 0)),
        ],
        out_specs=pl.BlockSpec((2, O), lambda b, r: (0, 0)),
        out_shape=jax.ShapeDtypeStruct((2, O), f32),
    )(g, w)

    g1 = gamma1.reshape(1, O)
    b1 = beta1.reshape(1, O)

    ymax, ymin, st2 = pl.pallas_call(
        functools.partial(_main_body, RB=RB, k=k, O=O, M=M),
        grid=(B, NB),
        in_specs=[
            pl.BlockSpec((1, RB, k, OP), lambda b, r: (b, r, 0, 0)),
            pl.BlockSpec((1, RB, O), lambda b, r: (b, r, 0)),
            pl.BlockSpec((2, O), lambda b, r: (0, 0)),
            pl.BlockSpec((1, O), lambda b, r: (0, 0)),
            pl.BlockSpec((1, O), lambda b, r: (0, 0)),
            pl.BlockSpec((O, O), lambda b, r: (0, 0)),
        ],
        out_specs=[
            pl.BlockSpec((1, RB, O), lambda b, r: (b, r, 0)),
            pl.BlockSpec((1, RB, O), lambda b, r: (b, r, 0)),
            pl.BlockSpec((2, O), lambda b, r: (0, 0)),
        ],
        out_shape=[
            jax.ShapeDtypeStruct((B, N, O), f32),
            jax.ShapeDtypeStruct((B, N, O), f32),
            jax.ShapeDtypeStruct((2, O), f32),
        ],
    )(g, w, st1, g1, b1, W2)

    g2 = gamma2.reshape(1, O)
    b2 = beta2.reshape(1, O)

    o = pl.pallas_call(
        functools.partial(_final_body, M=M),
        grid=(B, NB),
        in_specs=[
            pl.BlockSpec((1, RB, O), lambda b, r: (b, r, 0)),
            pl.BlockSpec((1, RB, O), lambda b, r: (b, r, 0)),
            pl.BlockSpec((2, O), lambda b, r: (0, 0)),
            pl.BlockSpec((1, O), lambda b, r: (0, 0)),
            pl.BlockSpec((1, O), lambda b, r: (0, 0)),
        ],
        out_specs=pl.BlockSpec((1, RB, O), lambda b, r: (b, r, 0)),
        out_shape=jax.ShapeDtypeStruct((B, N, O), f32),
    )(ymax, ymin, st2, g2, b2)

    return jnp.transpose(o, (0, 2, 1))


# speedup vs baseline: 11.4736x; 1.0067x over previous
import functools

import jax
import jax.numpy as jnp
from jax import lax
from jax.experimental import pallas as pl
from jax.experimental.pallas import tpu as pltpu
from jax.experimental.pallas import tpu_sc as plsc

KNN = 20
EPS_BN = 1e-5
NEG = -3.0e38


def _proj_body(xb_ref, w1_ref, u_ref, w_ref, *, C, O, OP):
    xcb = xb_ref[0]
    w1 = w1_ref[...]
    w1a = w1[:, :C]
    w1d = w1[:, C:] - w1a
    dn = (((0,), (1,)), ((), ()))
    ub = lax.dot_general(xcb, w1a, dn, preferred_element_type=jnp.float32)
    if OP > O:
        ub = jnp.concatenate(
            [ub, jnp.zeros((ub.shape[0], OP - O), jnp.float32)], axis=1)
    u_ref[0] = ub
    w_ref[0] = lax.dot_general(xcb, w1d, dn, preferred_element_type=jnp.float32)


def _topk_body(xb_ref, xf_ref, idx_ref, *, RB, N, k):
    b = pl.program_id(0)
    xcb = xb_ref[0]
    xcf = xf_ref[0]
    dn = (((0,), (0,)), ((), ()))
    g = lax.dot_general(xcb, xcf, dn, preferred_element_type=jnp.float32)
    sqf = jnp.sum(xcf * xcf, axis=0)
    d2 = 2.0 * g - sqf[None, :]
    iota = lax.broadcasted_iota(jnp.int32, (RB, N), 1)
    r = pl.program_id(1)
    selfc = r * RB + lax.broadcasted_iota(jnp.int32, (RB, 1), 0)
    cols = [selfc]
    d2 = jnp.where(iota == selfc, NEG, d2)
    for _ in range(k - 1):
        aj = jnp.argmax(d2, axis=1).astype(jnp.int32)[:, None]
        cols.append(aj)
        d2 = jnp.where(iota == aj, NEG, d2)
    idx = jnp.concatenate(cols, axis=1)
    idx_ref[0] = idx + b * N


def _stats1_body(g_ref, w_ref, st_ref, *, O):
    first = (pl.program_id(0) == 0) & (pl.program_id(1) == 0)
    y1 = g_ref[0][:, :, :O] + w_ref[0][:, None, :]
    s = jnp.sum(y1, axis=(0, 1))
    ss = jnp.sum(y1 * y1, axis=(0, 1))
    st = jnp.concatenate([s[None, :], ss[None, :]], axis=0)

    @pl.when(first)
    def _():
        st_ref[...] = st

    @pl.when(jnp.logical_not(first))
    def _():
        st_ref[...] = st_ref[...] + st


def _main_body(g_ref, w_ref, st1_ref, g1_ref, b1_ref, w2_ref,
               ymax_ref, ymin_ref, st2_ref, *, RB, k, O, M):
    first = (pl.program_id(0) == 0) & (pl.program_id(1) == 0)
    mean = st1_ref[0, :] * (1.0 / M)
    var = st1_ref[1, :] * (1.0 / M) - mean * mean
    inv = lax.rsqrt(var + EPS_BN)
    a1 = g1_ref[0] * inv
    c1 = b1_ref[0] - mean * a1
    y1 = g_ref[0][:, :, :O] + w_ref[0][:, None, :]
    z = y1 * a1[None, None, :] + c1[None, None, :]
    z = jnp.where(z >= 0, z, 0.2 * z)
    zf = z.reshape(RB * k, O)
    dn = (((1,), (1,)), ((), ()))
    y2 = lax.dot_general(zf, w2_ref[...], dn, preferred_element_type=jnp.float32)
    s = jnp.sum(y2, axis=0)
    ss = jnp.sum(y2 * y2, axis=0)
    st = jnp.concatenate([s[None, :], ss[None, :]], axis=0)
    y2r = y2.reshape(RB, k, O)
    ymax_ref[0] = jnp.max(y2r, axis=1)
    ymin_ref[0] = jnp.min(y2r, axis=1)

    @pl.when(first)
    def _():
        st2_ref[...] = st

    @pl.when(jnp.logical_not(first))
    def _():
        st2_ref[...] = st2_ref[...] + st


def _final_body(ymax_ref, ymin_ref, st2_ref, g2_ref, b2_ref, o_ref, *, M):
    mean = st2_ref[0, :] * (1.0 / M)
    var = st2_ref[1, :] * (1.0 / M) - mean * mean
    inv = lax.rsqrt(var + EPS_BN)
    a2 = g2_ref[0] * inv
    c2 = b2_ref[0] - mean * a2
    pick = jnp.where(a2[None, :] >= 0, ymax_ref[0], ymin_ref[0])
    v = pick * a2[None, :] + c2[None, :]
    o_ref[0] = jnp.transpose(jnp.where(v >= 0, v, 0.2 * v))


def _sc_gather(table, idx_flat, O):
    tot = idx_flat.shape[0]
    info = plsc.get_sparse_core_info()
    nc, ns = info.num_cores, info.num_subcores
    nw = nc * ns
    per_w = tot // nw
    ch = min(320, per_w)
    nch = per_w // ch
    mesh = plsc.VectorSubcoreMesh(core_axis_name="c", subcore_axis_name="s")

    @functools.partial(
        pl.kernel,
        mesh=mesh,
        out_type=jax.ShapeDtypeStruct((tot, O), jnp.float32),
        scratch_types=[
            pltpu.VMEM((per_w,), jnp.int32),
            pltpu.VMEM((ch, O), jnp.float32),
            pltpu.VMEM((ch, O), jnp.float32),
            pltpu.SemaphoreType.DMA,
            pltpu.SemaphoreType.DMA,
        ],
    )
    def gk(table_hbm, idx_hbm, out_hbm, idx_v, rows0, rows1, sem0, sem1):
        wid = lax.axis_index("s") * nc + lax.axis_index("c")
        base_w = wid * per_w
        pltpu.sync_copy(idx_hbm.at[pl.ds(base_w, per_w)], idx_v)

        def start(i, rows, sem):
            off = pl.multiple_of(i * ch, 8)
            pltpu.async_copy(table_hbm.at[idx_v.at[pl.ds(off, ch)]], rows, sem)

        def drain(rows, sem):
            pltpu.make_async_copy(table_hbm.at[pl.ds(0, ch)], rows, sem).wait()

        def wb(i, rows):
            off = pl.multiple_of(base_w + i * ch, 8)
            pltpu.sync_copy(rows, out_hbm.at[pl.ds(off, ch)])

        start(0, rows0, sem0)

        def body(t, carry):
            i0 = 2 * t
            start(i0 + 1, rows1, sem1)
            drain(rows0, sem0)
            wb(i0, rows0)

            @pl.when(t < (nch // 2) - 1)
            def _():
                start(i0 + 2, rows0, sem0)

            drain(rows1, sem1)
            wb(i0 + 1, rows1)
            return carry

        lax.fori_loop(0, nch // 2, body, 0)

    return gk(table, idx_flat)


def kernel(x, W1, gamma1, beta1, W2, gamma2, beta2):
    B, C, N = x.shape
    O = W1.shape[0]
    k = KNN
    RB = 256 if N % 256 == 0 else N
    NB = N // RB
    M = float(B * N * k)
    f32 = jnp.float32

    OP = O if O % 128 == 0 else ((O // 128) + 1) * 128

    u, w = pl.pallas_call(
        functools.partial(_proj_body, C=C, O=O, OP=OP),
        grid=(B, NB),
        in_specs=[
            pl.BlockSpec((1, C, RB), lambda b, r: (b, 0, r)),
            pl.BlockSpec((O, 2 * C), lambda b, r: (0, 0)),
        ],
        out_specs=[
            pl.BlockSpec((1, RB, OP), lambda b, r: (b, r, 0)),
            pl.BlockSpec((1, RB, O), lambda b, r: (b, r, 0)),
        ],
        out_shape=[
            jax.ShapeDtypeStruct((B, N, OP), f32),
            jax.ShapeDtypeStruct((B, N, O), f32),
        ],
    )(x, W1)

    idx = pl.pallas_call(
        functools.partial(_topk_body, RB=RB, N=N, k=k),
        grid=(B, NB),
        in_specs=[
            pl.BlockSpec((1, C, RB), lambda b, r: (b, 0, r)),
            pl.BlockSpec((1, C, N), lambda b, r: (b, 0, 0)),
        ],
        out_specs=pl.BlockSpec((1, RB, k), lambda b, r: (b, r, 0)),
        out_shape=jax.ShapeDtypeStruct((B, N, k), jnp.int32),
    )(x, x)

    g = _sc_gather(u.reshape(B * N, OP), idx.reshape(-1), OP).reshape(B, N, k, OP)

    st1 = pl.pallas_call(
        functools.partial(_stats1_body, O=O),
        grid=(B, NB),
        in_specs=[
            pl.BlockSpec((1, RB, k, OP), lambda b, r: (b, r, 0, 0)),
            pl.BlockSpec((1, RB, O), lambda b, r: (b, r, 0)),
        ],
        out_specs=pl.BlockSpec((2, O), lambda b, r: (0, 0)),
        out_shape=jax.ShapeDtypeStruct((2, O), f32),
    )(g, w)

    g1 = gamma1.reshape(1, O)
    b1 = beta1.reshape(1, O)

    ymax, ymin, st2 = pl.pallas_call(
        functools.partial(_main_body, RB=RB, k=k, O=O, M=M),
        grid=(B, NB),
        in_specs=[
            pl.BlockSpec((1, RB, k, OP), lambda b, r: (b, r, 0, 0)),
            pl.BlockSpec((1, RB, O), lambda b, r: (b, r, 0)),
            pl.BlockSpec((2, O), lambda b, r: (0, 0)),
            pl.BlockSpec((1, O), lambda b, r: (0, 0)),
            pl.BlockSpec((1, O), lambda b, r: (0, 0)),
            pl.BlockSpec((O, O), lambda b, r: (0, 0)),
        ],
        out_specs=[
            pl.BlockSpec((1, RB, O), lambda b, r: (b, r, 0)),
            pl.BlockSpec((1, RB, O), lambda b, r: (b, r, 0)),
            pl.BlockSpec((2, O), lambda b, r: (0, 0)),
        ],
        out_shape=[
            jax.ShapeDtypeStruct((B, N, O), f32),
            jax.ShapeDtypeStruct((B, N, O), f32),
            jax.ShapeDtypeStruct((2, O), f32),
        ],
    )(g, w, st1, g1, b1, W2)

    g2 = gamma2.reshape(1, O)
    b2 = beta2.reshape(1, O)

    o = pl.pallas_call(
        functools.partial(_final_body, M=M),
        grid=(B, NB),
        in_specs=[
            pl.BlockSpec((1, RB, O), lambda b, r: (b, r, 0)),
            pl.BlockSpec((1, RB, O), lambda b, r: (b, r, 0)),
            pl.BlockSpec((2, O), lambda b, r: (0, 0)),
            pl.BlockSpec((1, O), lambda b, r: (0, 0)),
            pl.BlockSpec((1, O), lambda b, r: (0, 0)),
        ],
        out_specs=pl.BlockSpec((1, O, RB), lambda b, r: (b, 0, r)),
        out_shape=jax.ShapeDtypeStruct((B, O, N), f32),
    )(ymax, ymin, st2, g2, b2)

    return o


# speedup vs baseline: 11.6297x; 1.0136x over previous
import functools

import jax
import jax.numpy as jnp
from jax import lax
from jax.experimental import pallas as pl
from jax.experimental.pallas import tpu as pltpu
from jax.experimental.pallas import tpu_sc as plsc

KNN = 20
EPS_BN = 1e-5
NEG = -3.0e38


def _proj_body(xb_ref, w1_ref, u_ref, w_ref, *, C, O, OP):
    xcb = xb_ref[0]
    w1 = w1_ref[...]
    w1a = w1[:, :C]
    w1d = w1[:, C:] - w1a
    dn = (((0,), (1,)), ((), ()))
    ub = lax.dot_general(xcb, w1a, dn, preferred_element_type=jnp.float32)
    if OP > O:
        ub = jnp.concatenate(
            [ub, jnp.zeros((ub.shape[0], OP - O), jnp.float32)], axis=1)
    u_ref[0] = ub
    w_ref[0] = lax.dot_general(xcb, w1d, dn, preferred_element_type=jnp.float32)


def _topk_body(xb_ref, xf_ref, idx_ref, *, RB, N, k):
    xcb = xb_ref[0]
    xcf = xf_ref[0]
    dn = (((0,), (0,)), ((), ()))
    g = lax.dot_general(xcb, xcf, dn, preferred_element_type=jnp.float32)
    sqf = jnp.sum(xcf * xcf, axis=0)
    d2 = 2.0 * g - sqf[None, :]
    iota = lax.broadcasted_iota(jnp.int32, (RB, N), 1)
    r = pl.program_id(0)
    selfc = r * RB + lax.broadcasted_iota(jnp.int32, (RB, 1), 0)
    cols = [selfc]
    d2 = jnp.where(iota == selfc, NEG, d2)
    for _ in range(k - 1):
        aj = jnp.argmax(d2, axis=1).astype(jnp.int32)[:, None]
        cols.append(aj)
        d2 = jnp.where(iota == aj, NEG, d2)
    idx = jnp.concatenate(cols, axis=1)
    idx_ref[...] = idx


def _stats1_body(g_ref, w_ref, st_ref, *, O):
    first = pl.program_id(0) == 0
    y1 = g_ref[...][:, :, :O] + w_ref[0][:, None, :]
    s = jnp.sum(y1, axis=(0, 1))
    ss = jnp.sum(y1 * y1, axis=(0, 1))
    st = jnp.concatenate([s[None, :], ss[None, :]], axis=0)

    @pl.when(first)
    def _():
        st_ref[...] = st

    @pl.when(jnp.logical_not(first))
    def _():
        st_ref[...] = st_ref[...] + st


def _main_body(g_ref, w_ref, st1_ref, g1_ref, b1_ref, w2_ref,
               ymax_ref, ymin_ref, st2_ref, *, RB, k, O, M):
    first = pl.program_id(0) == 0
    mean = st1_ref[0, :] * (1.0 / M)
    var = st1_ref[1, :] * (1.0 / M) - mean * mean
    inv = lax.rsqrt(var + EPS_BN)
    a1 = g1_ref[0] * inv
    c1 = b1_ref[0] - mean * a1
    y1 = g_ref[...][:, :, :O] + w_ref[0][:, None, :]
    z = y1 * a1[None, None, :] + c1[None, None, :]
    z = jnp.where(z >= 0, z, 0.2 * z)
    zf = z.reshape(RB * k, O)
    dn = (((1,), (1,)), ((), ()))
    y2 = lax.dot_general(zf, w2_ref[...], dn, preferred_element_type=jnp.float32)
    s = jnp.sum(y2, axis=0)
    ss = jnp.sum(y2 * y2, axis=0)
    st = jnp.concatenate([s[None, :], ss[None, :]], axis=0)
    y2r = y2.reshape(RB, k, O)
    ymax_ref[...] = jnp.max(y2r, axis=1)
    ymin_ref[...] = jnp.min(y2r, axis=1)

    @pl.when(first)
    def _():
        st2_ref[...] = st

    @pl.when(jnp.logical_not(first))
    def _():
        st2_ref[...] = st2_ref[...] + st


def _final_body(ymax_ref, ymin_ref, st2_ref, g2_ref, b2_ref, o_ref, *, M):
    mean = st2_ref[0, :] * (1.0 / M)
    var = st2_ref[1, :] * (1.0 / M) - mean * mean
    inv = lax.rsqrt(var + EPS_BN)
    a2 = g2_ref[0] * inv
    c2 = b2_ref[0] - mean * a2
    pick = jnp.where(a2[None, :] >= 0, ymax_ref[...], ymin_ref[...])
    v = pick * a2[None, :] + c2[None, :]
    o_ref[...] = jnp.transpose(jnp.where(v >= 0, v, 0.2 * v))


def _sc_gather(table, idx_flat, O):
    tot = idx_flat.shape[0]
    info = plsc.get_sparse_core_info()
    nc, ns = info.num_cores, info.num_subcores
    nw = nc * ns
    per_w = tot // nw
    ch = min(320, per_w)
    nch = per_w // ch
    mesh = plsc.VectorSubcoreMesh(core_axis_name="c", subcore_axis_name="s")

    @functools.partial(
        pl.kernel,
        mesh=mesh,
        out_type=jax.ShapeDtypeStruct((tot, O), jnp.float32),
        scratch_types=[
            pltpu.VMEM((per_w,), jnp.int32),
            pltpu.VMEM((ch, O), jnp.float32),
            pltpu.VMEM((ch, O), jnp.float32),
            pltpu.SemaphoreType.DMA,
            pltpu.SemaphoreType.DMA,
        ],
    )
    def gk(table_hbm, idx_hbm, out_hbm, idx_v, rows0, rows1, sem0, sem1):
        wid = lax.axis_index("s") * nc + lax.axis_index("c")
        base_w = wid * per_w
        pltpu.sync_copy(idx_hbm.at[pl.ds(base_w, per_w)], idx_v)

        def start(i, rows, sem):
            off = pl.multiple_of(i * ch, 8)
            pltpu.async_copy(table_hbm.at[idx_v.at[pl.ds(off, ch)]], rows, sem)

        def drain(rows, sem):
            pltpu.make_async_copy(table_hbm.at[pl.ds(0, ch)], rows, sem).wait()

        def wb(i, rows):
            off = pl.multiple_of(base_w + i * ch, 8)
            pltpu.sync_copy(rows, out_hbm.at[pl.ds(off, ch)])

        start(0, rows0, sem0)

        def body(t, carry):
            i0 = 2 * t
            start(i0 + 1, rows1, sem1)
            drain(rows0, sem0)
            wb(i0, rows0)

            @pl.when(t < (nch // 2) - 1)
            def _():
                start(i0 + 2, rows0, sem0)

            drain(rows1, sem1)
            wb(i0 + 1, rows1)
            return carry

        lax.fori_loop(0, nch // 2, body, 0)

    return gk(table, idx_flat)


def kernel(x, W1, gamma1, beta1, W2, gamma2, beta2):
    B, C, N = x.shape
    O = W1.shape[0]
    k = KNN
    RB = 256 if N % 256 == 0 else N
    NB = N // RB
    M = float(B * N * k)
    f32 = jnp.float32

    OP = O if O % 128 == 0 else ((O // 128) + 1) * 128

    u, w = pl.pallas_call(
        functools.partial(_proj_body, C=C, O=O, OP=OP),
        grid=(B, NB),
        in_specs=[
            pl.BlockSpec((1, C, RB), lambda b, r: (b, 0, r)),
            pl.BlockSpec((O, 2 * C), lambda b, r: (0, 0)),
        ],
        out_specs=[
            pl.BlockSpec((1, RB, OP), lambda b, r: (b, r, 0)),
            pl.BlockSpec((1, RB, O), lambda b, r: (b, r, 0)),
        ],
        out_shape=[
            jax.ShapeDtypeStruct((B, N, OP), f32),
            jax.ShapeDtypeStruct((B, N, O), f32),
        ],
    )(x, W1)

    gs = []
    for b in range(B):
        idx_b = pl.pallas_call(
            functools.partial(_topk_body, RB=RB, N=N, k=k),
            grid=(NB,),
            in_specs=[
                pl.BlockSpec((1, C, RB), lambda r, b=b: (b, 0, r)),
                pl.BlockSpec((1, C, N), lambda r, b=b: (b, 0, 0)),
            ],
            out_specs=pl.BlockSpec((RB, k), lambda r: (r, 0)),
            out_shape=jax.ShapeDtypeStruct((N, k), jnp.int32),
        )(x, x)
        gs.append(
            _sc_gather(u[b], idx_b.reshape(-1), OP).reshape(N, k, OP))

    g1 = gamma1.reshape(1, O)
    b1 = beta1.reshape(1, O)

    st1 = None
    for b in range(B):
        st1_b = pl.pallas_call(
            functools.partial(_stats1_body, O=O),
            grid=(NB,),
            in_specs=[
                pl.BlockSpec((RB, k, OP), lambda r: (r, 0, 0)),
                pl.BlockSpec((1, RB, O), lambda r, b=b: (b, r, 0)),
            ],
            out_specs=pl.BlockSpec((2, O), lambda r: (0, 0)),
            out_shape=jax.ShapeDtypeStruct((2, O), f32),
        )(gs[b], w)
        st1 = st1_b if st1 is None else st1 + st1_b

    ymaxs, ymins, st2 = [], [], None
    for b in range(B):
        ymax_b, ymin_b, st2_b = pl.pallas_call(
            functools.partial(_main_body, RB=RB, k=k, O=O, M=M),
            grid=(NB,),
            in_specs=[
                pl.BlockSpec((RB, k, OP), lambda r: (r, 0, 0)),
                pl.BlockSpec((1, RB, O), lambda r, b=b: (b, r, 0)),
                pl.BlockSpec((2, O), lambda r: (0, 0)),
                pl.BlockSpec((1, O), lambda r: (0, 0)),
                pl.BlockSpec((1, O), lambda r: (0, 0)),
                pl.BlockSpec((O, O), lambda r: (0, 0)),
            ],
            out_specs=[
                pl.BlockSpec((RB, O), lambda r: (r, 0)),
                pl.BlockSpec((RB, O), lambda r: (r, 0)),
                pl.BlockSpec((2, O), lambda r: (0, 0)),
            ],
            out_shape=[
                jax.ShapeDtypeStruct((N, O), f32),
                jax.ShapeDtypeStruct((N, O), f32),
                jax.ShapeDtypeStruct((2, O), f32),
            ],
        )(gs[b], w, st1, g1, b1, W2)
        ymaxs.append(ymax_b)
        ymins.append(ymin_b)
        st2 = st2_b if st2 is None else st2 + st2_b

    g2 = gamma2.reshape(1, O)
    b2 = beta2.reshape(1, O)

    os_ = []
    for b in range(B):
        os_.append(pl.pallas_call(
            functools.partial(_final_body, M=M),
            grid=(NB,),
            in_specs=[
                pl.BlockSpec((RB, O), lambda r: (r, 0)),
                pl.BlockSpec((RB, O), lambda r: (r, 0)),
                pl.BlockSpec((2, O), lambda r: (0, 0)),
                pl.BlockSpec((1, O), lambda r: (0, 0)),
                pl.BlockSpec((1, O), lambda r: (0, 0)),
            ],
            out_specs=pl.BlockSpec((O, RB), lambda r: (0, r)),
            out_shape=jax.ShapeDtypeStruct((O, N), f32),
        )(ymaxs[b], ymins[b], st2, g2, b2))

    return jnp.stack(os_, axis=0)
